# Initial kernel scaffold; baseline (speedup 1.0000x reference)
import jax
import jax.numpy as jnp
from jax.experimental import pallas as pl


def kernel(x, edge_index, W1, b1, W2, b2, fc1_w, fc1_b, fc2_w, fc2_b):
    raise NotImplementedError("write your pallas kernel here")


# speedup vs baseline: 13.8180x; 13.8180x over previous
import functools

import jax
import jax.numpy as jnp
from jax import lax
from jax.experimental import pallas as pl

N = 24
F = 128
E = 384
HID = 8


def _elu(v):
    return jnp.where(v > 0, v, jnp.exp(v) - 1.0)


def _dense_body(ei_ref, eiT_ref, x_ref, w10_ref, w11_ref, w12_ref, b1_ref,
                w20_ref, w21_ref, w22_ref, b2_ref, f1T_ref, f1b_ref,
                f2T_ref, f2b_ref, out_ref):
    f32 = jnp.float32

    dst_row = ei_ref[1:2, :]
    src_col = eiT_ref[:, 0:1]
    dd = lax.broadcasted_iota(jnp.int32, (N, E), 0)
    Md = (dd == jnp.broadcast_to(dst_row, (N, E))).astype(f32)
    ss = lax.broadcasted_iota(jnp.int32, (E, N), 1)
    MsT = (ss == jnp.broadcast_to(src_col, (E, N))).astype(f32)
    C = jnp.dot(Md, MsT, preferred_element_type=f32)

    deg = jnp.sum(C, axis=0, keepdims=True)
    dinv = jnp.where(deg > 0, lax.rsqrt(jnp.where(deg > 0, deg, 1.0)), 0.0)
    i0 = lax.broadcasted_iota(jnp.int32, (N, N), 0)
    i1 = lax.broadcasted_iota(jnp.int32, (N, N), 1)
    eye = (i0 == i1).astype(f32)
    dinv_col = jnp.sum(eye * jnp.broadcast_to(dinv, (N, N)), axis=1,
                       keepdims=True)
    S = -(C * dinv_col) * dinv

    x = x_ref[:, :]
    P0 = jnp.dot(x, w10_ref[:, :], preferred_element_type=f32)
    P1 = jnp.dot(x, w11_ref[:, :], preferred_element_type=f32)
    P2 = jnp.dot(x, w12_ref[:, :], preferred_element_type=f32)
    SP1 = jnp.dot(S, P1, preferred_element_type=f32)
    SSP2 = jnp.dot(S, jnp.dot(S, P2, preferred_element_type=f32),
                   preferred_element_type=f32)
    h = _elu(P0 + SP1 + 2.0 * SSP2 - P2 + b1_ref[:, :])

    Q0 = jnp.dot(h, w20_ref[:, :], preferred_element_type=f32)
    Q1 = jnp.dot(h, w21_ref[:, :], preferred_element_type=f32)
    Q2 = jnp.dot(h, w22_ref[:, :], preferred_element_type=f32)
    SQ1 = jnp.dot(S, Q1, preferred_element_type=f32)
    SSQ2 = jnp.dot(S, jnp.dot(S, Q2, preferred_element_type=f32),
                   preferred_element_type=f32)
    g = _elu(Q0 + SQ1 + 2.0 * SSQ2 - Q2 + b2_ref[:, :])

    NH = N * HID
    rk = lax.broadcasted_iota(jnp.int32, (NH, N), 0)
    rn = lax.broadcasted_iota(jnp.int32, (NH, N), 1)
    Rep = ((rk // HID) == rn).astype(f32)
    Gr = jnp.dot(Rep, g, preferred_element_type=f32)
    fk = lax.broadcasted_iota(jnp.int32, (NH, HID), 0)
    ff = lax.broadcasted_iota(jnp.int32, (NH, HID), 1)
    sel = ((fk % HID) == ff).astype(f32)
    gcol = jnp.sum(Gr * sel, axis=1, keepdims=True)

    Wc2 = jnp.dot(f2T_ref[:, :], f1T_ref[:, :], preferred_element_type=f32)
    bc = jnp.dot(f2T_ref[:, :], f1b_ref[:, :], preferred_element_type=f32) \
        + f2b_ref[:, :]
    v2 = jnp.dot(Wc2, gcol, preferred_element_type=f32) + bc

    m = jnp.max(v2)
    lse = m + jnp.log(jnp.sum(jnp.exp(v2 - m)))
    o = v2 - lse
    r0 = jnp.sum(jnp.where(lax.broadcasted_iota(jnp.int32, (2, 1), 0) == 0,
                           o, 0.0))
    r1 = jnp.sum(jnp.where(lax.broadcasted_iota(jnp.int32, (2, 1), 0) == 1,
                           o, 0.0))
    cmask = lax.broadcasted_iota(jnp.int32, (1, 2), 1)
    out_ref[:, :] = jnp.where(cmask == 0, r0, r1)


@functools.partial(jax.jit, static_argnames=("interpret",))
def _run(x, edge_index, W1, b1, W2, b2, fc1_w, fc1_b, fc2_w, fc2_b,
         interpret=False):
    eiT = edge_index.T
    out = pl.pallas_call(
        _dense_body,
        out_shape=jax.ShapeDtypeStruct((1, 2), jnp.float32),
        interpret=interpret,
    )(edge_index, eiT, x, W1[0], W1[1], W1[2], b1.reshape(1, HID),
      W2[0], W2[1], W2[2], b2.reshape(1, HID),
      fc1_w.T, fc1_b.reshape(-1, 1), fc2_w.T, fc2_b.reshape(-1, 1))
    return out


def kernel(x, edge_index, W1, b1, W2, b2, fc1_w, fc1_b, fc2_w, fc2_b):
    return _run(x, edge_index, W1, b1, W2, b2, fc1_w, fc1_b, fc2_w, fc2_b)
